# baseline (speedup 1.0000x reference)
import functools

import jax
import jax.numpy as jnp
from jax import lax
from jax.experimental import pallas as pl
from jax.experimental.pallas import tpu as pltpu
from jax.experimental.pallas import tpu_sc as plsc

VOCAB = 1000000
DIM = 64
BATCH = 4096
HIST = 200
B = BATCH * HIST

NC = 2
NS = 16
NW = NC * NS
CHUNK = 256
CPH = BATCH // CHUNK
NSLAB = HIST * CPH
SPW = NSLAB // NW
HD = DIM * BATCH

_mesh = plsc.VectorSubcoreMesh(core_axis_name="c", subcore_axis_name="s")


@functools.partial(
    pl.kernel,
    mesh=_mesh,
    out_type=jax.ShapeDtypeStruct((HIST * DIM * BATCH,), jnp.float32),
    scratch_types=[
        pltpu.VMEM((2, CHUNK), jnp.int32),
        pltpu.VMEM((2, CHUNK, DIM), jnp.float32),
        pltpu.VMEM((2, DIM * CHUNK), jnp.float32),
        pltpu.SemaphoreType.DMA,
        pltpu.SemaphoreType.DMA,
        pltpu.SemaphoreType.DMA,
    ],
    compiler_params=pltpu.CompilerParams(
        use_tc_tiling_on_sc=False, needs_layout_passes=False),
)
def _gather_t(table_hbm, idx_hbm, out_hbm, idx_v, rows_v, vout, sem_l, sem_g, sem_w):
    wid = lax.axis_index("s") * NC + lax.axis_index("c")
    base_slab = wid * SPW

    def idx_off(s):
        sid = base_slab + s
        h = sid // CPH
        bb = sid % CPH
        return h * BATCH + bb * CHUNK

    def out_off(s):
        sid = base_slab + s
        h = sid // CPH
        bb = sid % CPH
        return h * HD + bb * CHUNK

    def load_idx(s, b):
        pltpu.sync_copy(idx_hbm.at[pl.ds(idx_off(s), CHUNK)], idx_v.at[b])

    def start_gather(b):
        pltpu.async_copy(table_hbm.at[idx_v.at[b]], rows_v.at[b], sem_g)

    def wait_gather(b):
        pltpu.make_async_copy(
            table_hbm.at[idx_v.at[b]], rows_v.at[b], sem_g).wait()

    def start_writes(s, b):
        o = out_off(s)
        for d in range(DIM):
            pltpu.async_copy(
                vout.at[b, pl.ds(d * CHUNK, CHUNK)],
                out_hbm.at[pl.ds(o + d * BATCH, CHUNK)],
                sem_w,
            )

    def drain_writes(b):
        for d in range(DIM):
            pltpu.make_async_copy(
                vout.at[b, pl.ds(d * CHUNK, CHUNK)],
                out_hbm.at[pl.ds(d * BATCH, CHUNK)],
                sem_w,
            ).wait()

    iota16 = lax.iota(jnp.int32, 16) * CHUNK

    def transpose(b):
        def body(bb, carry):
            for m in range(DIM // 16):
                vals = rows_v[b, bb, pl.ds(16 * m, 16)]
                fidx = iota16 + (16 * m * CHUNK + bb)
                plsc.store_scatter(vout.at[b], [fidx], vals)
            return carry

        lax.fori_loop(0, CHUNK, body, 0, unroll=4)

    load_idx(0, 0)
    start_gather(0)

    def loop(s, carry):
        b0 = s % 2
        b1 = 1 - b0
        wait_gather(b0)

        @pl.when(s < SPW - 1)
        def _():
            load_idx(s + 1, b1)
            start_gather(b1)

        @pl.when(s >= 2)
        def _():
            drain_writes(b0)

        transpose(b0)
        start_writes(s, b0)
        return carry

    lax.fori_loop(0, SPW, loop, 0)
    drain_writes(0)
    drain_writes(1)


def kernel(inputs, table):
    idx = inputs.T.reshape(B)
    flat = _gather_t(table, idx)
    return flat.reshape(HIST, DIM, BATCH).transpose(2, 0, 1)


# speedup vs baseline: 1.1761x; 1.1761x over previous
import functools

import jax
import jax.numpy as jnp
from jax import lax
from jax.experimental import pallas as pl
from jax.experimental.pallas import tpu as pltpu
from jax.experimental.pallas import tpu_sc as plsc

VOCAB = 1000000
DIM = 64
BATCH = 4096
HIST = 200
B = BATCH * HIST

NC = 2
NS = 16
NW = NC * NS
CHUNK = 256
CPH = BATCH // CHUNK
NSLAB = HIST * CPH
SPW = NSLAB // NW
HD = DIM * BATCH

_mesh = plsc.VectorSubcoreMesh(core_axis_name="c", subcore_axis_name="s")


@functools.partial(
    pl.kernel,
    mesh=_mesh,
    out_type=jax.ShapeDtypeStruct((HIST * DIM * BATCH,), jnp.float32),
    scratch_types=[
        pltpu.VMEM((2, CHUNK), jnp.int32),
        pltpu.VMEM((2, CHUNK, DIM), jnp.float32),
        pltpu.VMEM((2, DIM * CHUNK), jnp.float32),
        pltpu.SemaphoreType.DMA,
        pltpu.SemaphoreType.DMA,
        pltpu.SemaphoreType.DMA,
    ],
    compiler_params=pltpu.CompilerParams(
        use_tc_tiling_on_sc=False, needs_layout_passes=False),
)
def _gather_t(table_hbm, idx_hbm, out_hbm, idx_v, rows_v, vout, sem_l, sem_g, sem_w):
    wid = lax.axis_index("s") * NC + lax.axis_index("c")
    base_slab = wid * SPW

    def idx_off(s):
        sid = base_slab + s
        h = sid // CPH
        bb = sid % CPH
        return h * BATCH + bb * CHUNK

    def out_off(s):
        sid = base_slab + s
        h = sid // CPH
        bb = sid % CPH
        return h * HD + bb * CHUNK

    def load_idx(s, b):
        pltpu.sync_copy(idx_hbm.at[pl.ds(idx_off(s), CHUNK)], idx_v.at[b])

    def start_gather(b):
        pltpu.async_copy(table_hbm.at[idx_v.at[b]], rows_v.at[b], sem_g)

    def wait_gather(b):
        pltpu.make_async_copy(
            table_hbm.at[idx_v.at[b]], rows_v.at[b], sem_g).wait()

    def start_writes(s, b):
        o = out_off(s)
        for d in range(DIM):
            pltpu.async_copy(
                vout.at[b, pl.ds(d * CHUNK, CHUNK)],
                out_hbm.at[pl.ds(o + d * BATCH, CHUNK)],
                sem_w,
            )

    def drain_writes(b):
        for d in range(DIM):
            pltpu.make_async_copy(
                vout.at[b, pl.ds(d * CHUNK, CHUNK)],
                out_hbm.at[pl.ds(d * BATCH, CHUNK)],
                sem_w,
            ).wait()

    iota16 = lax.iota(jnp.int32, 16) * CHUNK

    def transpose(b):
        @plsc.parallel_loop(0, CHUNK, 1, unroll=8)
        def body(bb):
            for m in range(DIM // 16):
                vals = rows_v[b, bb, pl.ds(16 * m, 16)]
                fidx = iota16 + (16 * m * CHUNK + bb)
                plsc.store_scatter(vout.at[b], [fidx], vals)

    load_idx(0, 0)
    start_gather(0)

    def loop(s, carry):
        b0 = s % 2
        b1 = 1 - b0
        wait_gather(b0)

        @pl.when(s < SPW - 1)
        def _():
            load_idx(s + 1, b1)
            start_gather(b1)

        @pl.when(s >= 2)
        def _():
            drain_writes(b0)

        transpose(b0)
        start_writes(s, b0)
        return carry

    lax.fori_loop(0, SPW, loop, 0)
    drain_writes(0)
    drain_writes(1)


def kernel(inputs, table):
    idx = inputs.T.reshape(B)
    flat = _gather_t(table, idx)
    return flat.reshape(HIST, DIM, BATCH).transpose(2, 0, 1)


# speedup vs baseline: 1.5438x; 1.3127x over previous
import functools

import jax
import jax.numpy as jnp
from jax import lax
from jax.experimental import pallas as pl
from jax.experimental.pallas import tpu as pltpu
from jax.experimental.pallas import tpu_sc as plsc

VOCAB = 1000000
DIM = 64
BATCH = 4096
HIST = 200
B = BATCH * HIST

NC = 2
NS = 16
NW = NC * NS
B_PER_W = B // NW
CHUNK = 800
NCHUNK = B_PER_W // CHUNK
NBUF = 2
NPAIR = NCHUNK // NBUF

_mesh = plsc.VectorSubcoreMesh(core_axis_name="c", subcore_axis_name="s")


@functools.partial(
    pl.kernel,
    mesh=_mesh,
    out_type=jax.ShapeDtypeStruct((B, DIM), jnp.float32),
    scratch_types=[
        pltpu.VMEM((NBUF, CHUNK), jnp.int32),
        pltpu.VMEM((NBUF, CHUNK, DIM), jnp.float32),
        pltpu.SemaphoreType.DMA,
        pltpu.SemaphoreType.DMA,
        pltpu.SemaphoreType.DMA,
    ],
    compiler_params=pltpu.CompilerParams(use_tc_tiling_on_sc=False),
)
def _gather(table_hbm, idx_hbm, out_hbm, idx_v, rows_v, sem_l, sem_g, sem_w):
    wid = lax.axis_index("s") * NC + lax.axis_index("c")
    base = wid * B_PER_W

    def load(i, b):
        pltpu.async_copy(
            idx_hbm.at[pl.ds(base + i * CHUNK, CHUNK)], idx_v.at[b], sem_l)

    def wait_load(b):
        pltpu.make_async_copy(
            idx_hbm.at[pl.ds(base, CHUNK)], idx_v.at[b], sem_l).wait()

    def gather(b):
        return pltpu.async_copy(table_hbm.at[idx_v.at[b]], rows_v.at[b], sem_g)

    def write(i, b):
        pltpu.async_copy(
            rows_v.at[b], out_hbm.at[pl.ds(base + i * CHUNK, CHUNK)], sem_w)

    def wait_write(b):
        pltpu.make_async_copy(
            rows_v.at[b], out_hbm.at[pl.ds(base, CHUNK)], sem_w).wait()

    for b in range(NBUF):
        load(b, b)
    for b in range(NBUF):
        wait_load(b)
        gather(b).wait()
        write(b, b)
        load(b + NBUF, b)

    def body(k, carry):
        for b in range(NBUF):
            i = k * NBUF + b
            wait_write(b)
            wait_load(b)
            gather(b).wait()
            write(i, b)
            load(i + NBUF, b)
        return carry

    lax.fori_loop(1, NPAIR - 1, body, 0)

    for b in range(NBUF):
        i = NCHUNK - NBUF + b
        wait_write(b)
        wait_load(b)
        gather(b).wait()
        write(i, b)
    for b in range(NBUF):
        wait_write(b)


def kernel(inputs, table):
    idx = inputs.reshape(B)
    out = _gather(table, idx)
    return out.reshape(BATCH, HIST, DIM)
